# baseline (speedup 1.0000x reference)
import jax
import jax.numpy as jnp
from jax import lax
from jax.experimental import pallas as pl
from jax.experimental.pallas import tpu as pltpu
from jax.experimental.pallas import tpu_sc as plsc

BATCH = 16384
EMB = 64
CDIM = 128
NC = 2
NS = 16
NW = NC * NS
BPW = BATCH // NW
CHUNK = 128
NCHUNK = BPW // CHUNK


NSLOT = 4


def _sc_gather_body(u_tab, c_tab, iu_hbm, ii_hbm, out_u, out_c,
                    idx_u, idx_c, buf,
                    sg0, sg1, sg2, sg3, sw0, sw1, sw2, sw3):
    sem_g = [sg0, sg1, sg2, sg3]
    sem_w = [sw0, sw1, sw2, sw3]
    wid = lax.axis_index("s") * NC + lax.axis_index("c")
    base = wid * BPW
    rbase = wid * NCHUNK
    pltpu.sync_copy(iu_hbm.at[pl.ds(rbase, NCHUNK)], idx_u)
    pltpu.sync_copy(ii_hbm.at[pl.ds(rbase, NCHUNK)], idx_c)
    tasks = []
    for j in range(NCHUNK):
        tasks.append((u_tab, idx_u, out_u, j))
        tasks.append((c_tab, idx_c, out_c, j))
    nt = len(tasks)
    gh = [None] * nt
    wh = [None] * nt
    for k, (tab, idx, out, j) in enumerate(tasks):
        s = k % NSLOT
        if k >= NSLOT:
            wh[k - NSLOT].wait()
        gh[k] = pltpu.async_copy(tab.at[idx.at[j]], buf.at[s], sem_g[s])
        if k >= 1:
            pk = k - 1
            ptab, pidx, pout, pj = tasks[pk]
            gh[pk].wait()
            wh[pk] = pltpu.async_copy(
                buf.at[pk % NSLOT], pout.at[pl.ds(base + pj * CHUNK, CHUNK)],
                sem_w[pk % NSLOT])
    gh[nt - 1].wait()
    ltab, lidx, lout, lj = tasks[nt - 1]
    wh[nt - 1] = pltpu.async_copy(
        buf.at[(nt - 1) % NSLOT], lout.at[pl.ds(base + lj * CHUNK, CHUNK)],
        sem_w[(nt - 1) % NSLOT])
    for k in range(nt - NSLOT, nt):
        wh[k].wait()


@jax.jit
def _sc_gather(user_pairs, item_content, iu, ii):
    mesh = plsc.VectorSubcoreMesh(core_axis_name="c", subcore_axis_name="s")
    return pl.kernel(
        _sc_gather_body,
        out_type=(
            jax.ShapeDtypeStruct((BATCH, 2 * EMB), jnp.float32),
            jax.ShapeDtypeStruct((BATCH, CDIM), jnp.float32),
        ),
        mesh=mesh,
        scratch_types=[
            pltpu.VMEM((NCHUNK, CHUNK), jnp.int32),
            pltpu.VMEM((NCHUNK, CHUNK), jnp.int32),
            pltpu.VMEM((NSLOT, CHUNK, CDIM), jnp.float32),
        ] + [pltpu.SemaphoreType.DMA] * (2 * NSLOT),
    )(user_pairs, item_content, iu, ii)


def _tc_body(u_ref, c_ref, w_ref, b_ref, s_ref, o_ref):
    meta = lax.dot_general(c_ref[...], w_ref[...],
                           (((1,), (1,)), ((), ())),
                           preferred_element_type=jnp.float32)
    meta = (meta + b_ref[...]) / 5.0
    sel = s_ref[...]
    u = u_ref[:, 0, :] * (1.0 - sel) + u_ref[:, 1, :] * sel
    o_ref[...] = jnp.sum(u * meta, axis=1)[None, None, :]


@jax.jit
def _tc_compute(u_g, c_g, W, b2, sel):
    blk = 2048
    grid = BATCH // blk
    out = pl.pallas_call(
        _tc_body,
        grid=(grid,),
        in_specs=[
            pl.BlockSpec((blk, 2, EMB), lambda i: (i, 0, 0)),
            pl.BlockSpec((blk, CDIM), lambda i: (i, 0)),
            pl.BlockSpec((EMB, CDIM), lambda i: (0, 0)),
            pl.BlockSpec((1, EMB), lambda i: (0, 0)),
            pl.BlockSpec((blk, 1), lambda i: (i, 0)),
        ],
        out_specs=pl.BlockSpec((1, 1, blk), lambda i: (i, 0, 0)),
        out_shape=jax.ShapeDtypeStruct((grid, 1, blk), jnp.float32),
    )(u_g, c_g, W, b2, sel)
    return out.reshape(BATCH)


def kernel(batch_u, batch_i, user_emb, item_emb, item_content, W, b):
    bu = batch_u.astype(jnp.int32)
    iu = (bu >> 1).reshape(BATCH // CHUNK, CHUNK)
    ii = batch_i.astype(jnp.int32).reshape(BATCH // CHUNK, CHUNK)
    sel = (bu & 1).astype(jnp.float32).reshape(BATCH, 1)
    user_pairs = user_emb.reshape(-1, 2 * EMB)
    u_g, c_g = _sc_gather(user_pairs, item_content, iu, ii)
    return _tc_compute(u_g.reshape(BATCH, 2, EMB), c_g, W,
                       b.reshape(1, EMB), sel)


# speedup vs baseline: 1.4379x; 1.4379x over previous
import jax
import jax.numpy as jnp
from jax import lax
from jax.experimental import pallas as pl
from jax.experimental.pallas import tpu as pltpu
from jax.experimental.pallas import tpu_sc as plsc

BATCH = 16384
EMB = 64
CDIM = 128
NC = 2
NS = 16
NW = NC * NS
BPW = BATCH // NW
CHUNK = 128
NCHUNK = BPW // CHUNK


NSLOT = 4


def _sc_gather_body(u_tab, c_tab, iu_hbm, ii_hbm, out_u, out_c,
                    idx_u, idx_c, buf,
                    sg0, sg1, sg2, sg3, sw0, sw1, sw2, sw3):
    sem_g = [sg0, sg1, sg2, sg3]
    sem_w = [sw0, sw1, sw2, sw3]
    wid = lax.axis_index("s") * NC + lax.axis_index("c")
    base = wid * BPW
    rbase = wid * NCHUNK
    pltpu.sync_copy(iu_hbm.at[pl.ds(rbase, NCHUNK)], idx_u)
    pltpu.sync_copy(ii_hbm.at[pl.ds(rbase, NCHUNK)], idx_c)
    tasks = []
    for j in range(NCHUNK):
        tasks.append((u_tab, idx_u, out_u, j))
        tasks.append((c_tab, idx_c, out_c, j))
    nt = len(tasks)
    gh = [None] * nt
    wh = [None] * nt
    for k, (tab, idx, out, j) in enumerate(tasks):
        s = k % NSLOT
        if k >= NSLOT:
            wh[k - NSLOT].wait()
        gh[k] = pltpu.async_copy(tab.at[idx.at[j]], buf.at[s], sem_g[s])
        if k >= 1:
            pk = k - 1
            ptab, pidx, pout, pj = tasks[pk]
            gh[pk].wait()
            wh[pk] = pltpu.async_copy(
                buf.at[pk % NSLOT], pout.at[pl.ds(base + pj * CHUNK, CHUNK)],
                sem_w[pk % NSLOT])
    gh[nt - 1].wait()
    ltab, lidx, lout, lj = tasks[nt - 1]
    wh[nt - 1] = pltpu.async_copy(
        buf.at[(nt - 1) % NSLOT], lout.at[pl.ds(base + lj * CHUNK, CHUNK)],
        sem_w[(nt - 1) % NSLOT])
    for k in range(nt - NSLOT, nt):
        wh[k].wait()


@jax.jit
def _sc_gather(user_pairs, item_content, iu, ii):
    mesh = plsc.VectorSubcoreMesh(core_axis_name="c", subcore_axis_name="s")
    return pl.kernel(
        _sc_gather_body,
        out_type=(
            jax.ShapeDtypeStruct((BATCH, 2 * EMB), jnp.float32),
            jax.ShapeDtypeStruct((BATCH, CDIM), jnp.float32),
        ),
        mesh=mesh,
        scratch_types=[
            pltpu.VMEM((NCHUNK, CHUNK), jnp.int32),
            pltpu.VMEM((NCHUNK, CHUNK), jnp.int32),
            pltpu.VMEM((NSLOT, CHUNK, CDIM), jnp.float32),
        ] + [pltpu.SemaphoreType.DMA] * (2 * NSLOT),
    )(user_pairs, item_content, iu, ii)


def _tc_body(u_ref, c_ref, w2_ref, b2_ref, m_ref, o_ref):
    meta2 = lax.dot_general(c_ref[...], w2_ref[...],
                            (((1,), (0,)), ((), ())),
                            preferred_element_type=jnp.float32)
    meta2 = meta2 + b2_ref[...]
    o_ref[...] = jnp.sum(u_ref[...] * meta2 * m_ref[...], axis=1)[None, None, :]


@jax.jit
def _tc_compute(u_g, c_g, W2, b2, msk):
    blk = 2048
    grid = BATCH // blk
    out = pl.pallas_call(
        _tc_body,
        grid=(grid,),
        in_specs=[
            pl.BlockSpec((blk, 2 * EMB), lambda i: (i, 0)),
            pl.BlockSpec((blk, CDIM), lambda i: (i, 0)),
            pl.BlockSpec((CDIM, 2 * EMB), lambda i: (0, 0)),
            pl.BlockSpec((1, 2 * EMB), lambda i: (0, 0)),
            pl.BlockSpec((blk, 2 * EMB), lambda i: (i, 0)),
        ],
        out_specs=pl.BlockSpec((1, 1, blk), lambda i: (i, 0, 0)),
        out_shape=jax.ShapeDtypeStruct((grid, 1, blk), jnp.float32),
    )(u_g, c_g, W2, b2, msk)
    return out.reshape(BATCH)


def kernel(batch_u, batch_i, user_emb, item_emb, item_content, W, b):
    bu = batch_u.astype(jnp.int32)
    iu = (bu >> 1).reshape(BATCH // CHUNK, CHUNK)
    ii = batch_i.astype(jnp.int32).reshape(BATCH // CHUNK, CHUNK)
    user_pairs = user_emb.reshape(-1, 2 * EMB)
    u_g, c_g = _sc_gather(user_pairs, item_content, iu, ii)
    W2 = jnp.concatenate([W.T, W.T], axis=1)
    b2 = jnp.concatenate([b, b]).reshape(1, 2 * EMB)
    par = (bu & 1).reshape(BATCH, 1)
    msk = jnp.where((jnp.arange(2 * EMB, dtype=jnp.int32)[None, :] >> 6)
                    == par, jnp.float32(0.2), jnp.float32(0.0))
    return _tc_compute(u_g, c_g, W2, b2, msk)
